# baseline (speedup 1.0000x reference)
import functools

import jax
import jax.numpy as jnp
from jax import lax
from jax.experimental import pallas as pl
from jax.experimental.pallas import tpu as pltpu
from jax.experimental.pallas import tpu_sc as plsc

_NG, _NC, _NGO, _E, _D = 50000, 50000, 10000, 150000, 128
_NSUB = 16
_NCORE = 2
_CH = 256
_K = 40
_KH = _K // 2
_EPAD = _NSUB * _K * _CH
_R = 1000

_RELS = ("g2c", "c2g", "g2go", "go2g")
_SRC_T = {"g2c": "gene", "c2g": "cell", "g2go": "gene", "go2g": "gotem"}
_DST_T = {"g2c": "cell", "c2g": "gene", "g2go": "gotem", "go2g": "gene"}
_NT = {"gene": _NG, "cell": _NC, "gotem": _NGO}


def _npad(n):
    return ((n + 1 + 127) // 128) * 128


_NPADS = {t: _npad(n) for t, n in _NT.items()}
_ZROWS = max(_NPADS.values()) // _NSUB
_ZBROWS = _ZROWS // 2

_MESH = functools.partial(
    plsc.VectorSubcoreMesh,
    core_axis_name="c", subcore_axis_name="s",
    num_cores=_NCORE, num_subcores=_NSUB,
)


def _degree_call(idx_arrs, npads, ones8, zeros8):
    n = len(idx_arrs)

    def body(*refs):
        idx_refs = refs[:n]
        ones_h, zeros_h = refs[n], refs[n + 1]
        out_refs = refs[n + 2: 2 * n + 2]
        idxbuf, onesbuf, acc = refs[2 * n + 2], refs[2 * n + 3], refs[2 * n + 4]
        cid = lax.axis_index("c")
        sid = lax.axis_index("s")
        pltpu.sync_copy(ones_h, onesbuf)
        for t in range(n):
            rows = npads[t] // _NSUB

            @pl.when(cid == (t % _NCORE))
            def _(t=t, rows=rows):
                pltpu.sync_copy(zeros_h.at[pl.ds(0, rows)],
                                acc.at[pl.ds(sid * rows, rows)])
                pltpu.sync_copy(idx_refs[t].at[sid], idxbuf)
                plsc.subcore_barrier()

                def step(j, carry):
                    pltpu.sync_copy(onesbuf, acc.at[idxbuf.at[j]], add=True)
                    return carry

                lax.fori_loop(0, _K, step, 0)
                plsc.subcore_barrier()
                pltpu.sync_copy(acc.at[pl.ds(sid * rows, rows)],
                                out_refs[t].at[pl.ds(sid * rows, rows)])
                plsc.subcore_barrier()

    out_type = [jax.ShapeDtypeStruct((p, 8), jnp.float32) for p in npads]
    f = pl.kernel(
        body,
        out_type=out_type,
        mesh=_MESH(),
        scratch_types=[
            pltpu.VMEM((_K, _CH), jnp.int32),
            pltpu.VMEM((_CH, 8), jnp.float32),
            pltpu.VMEM_SHARED((max(npads), 8), jnp.float32),
        ],
        compiler_params=pltpu.CompilerParams(use_tc_tiling_on_sc=False),
        name="rgcn_degrees",
    )
    return f(*idx_arrs, ones8, zeros8)


def _sc_layer(specs, zeros32):
    nrel = len(specs)
    npads = [s[3] for s in specs]

    def body(*refs):
        z_refs = refs[0:nrel]
        g_refs = refs[nrel:2 * nrel]
        d_refs = refs[2 * nrel:3 * nrel]
        zeros_h = refs[3 * nrel]
        out_refs = refs[3 * nrel + 1: 4 * nrel + 1]
        idxbuf, dstbuf, rb0, rb1, acc, g0, g1 = refs[4 * nrel + 1:]
        cid = lax.axis_index("c")
        sid = lax.axis_index("s")
        for t in range(nrel):
            rows = npads[t] // _NSUB
            z_ref = z_refs[t]
            for gl in range(2):
                c = cid * 2 + gl
                for zo in range(0, rows, _ZBROWS):
                    zr = min(_ZBROWS, rows - zo)
                    pltpu.sync_copy(zeros_h.at[pl.ds(0, zr)],
                                    acc.at[pl.ds(sid * rows + zo, zr)])
                plsc.subcore_barrier()
                for h in range(2):
                    pltpu.sync_copy(g_refs[t].at[c, sid, h], idxbuf)
                    pltpu.sync_copy(d_refs[t].at[sid, h], dstbuf)
                    pltpu.async_copy(z_ref.at[idxbuf.at[0]], rb0, g0)

                    def round_(r, carry, t=t, z_ref=z_ref):
                        j = 2 * r
                        pltpu.async_copy(z_ref.at[idxbuf.at[j + 1]], rb1, g1)
                        pltpu.make_async_copy(z_ref.at[idxbuf.at[j]], rb0,
                                              g0).wait()
                        pltpu.sync_copy(rb0, acc.at[dstbuf.at[j]], add=True)

                        @pl.when(j + 2 < _KH)
                        def _():
                            pltpu.async_copy(z_ref.at[idxbuf.at[j + 2]],
                                             rb0, g0)
                        pltpu.make_async_copy(z_ref.at[idxbuf.at[j + 1]],
                                              rb1, g1).wait()
                        pltpu.sync_copy(rb1, acc.at[dstbuf.at[j + 1]],
                                        add=True)
                        return carry

                    lax.fori_loop(0, _KH // 2, round_, 0)
                plsc.subcore_barrier()
                pltpu.sync_copy(acc.at[pl.ds(sid * rows, rows)],
                                out_refs[t].at[c, pl.ds(sid * rows, rows)])
                plsc.subcore_barrier()

    out_type = [jax.ShapeDtypeStruct((4, p, 32), jnp.float32) for p in npads]
    f = pl.kernel(
        body,
        out_type=out_type,
        mesh=_MESH(),
        scratch_types=[
            pltpu.VMEM((_KH, _CH), jnp.int32),
            pltpu.VMEM((_KH, _CH), jnp.int32),
            pltpu.VMEM((_CH, 32), jnp.float32),
            pltpu.VMEM((_CH, 32), jnp.float32),
            pltpu.VMEM_SHARED((max(npads), 32), jnp.float32),
            pltpu.SemaphoreType.DMA,
            pltpu.SemaphoreType.DMA,
        ],
        compiler_params=pltpu.CompilerParams(use_tc_tiling_on_sc=False),
        name="rgcn_sc_layer_%d" % nrel,
    )
    ins = []
    for grp in range(3):
        ins += [s[grp] for s in specs]
    outs = f(*ins, zeros32)
    return outs if isinstance(outs, (list, tuple)) else [outs]


def _inv_sqrt_deg(d):
    return jnp.where(d > 0, lax.rsqrt(jnp.maximum(d, 1.0)), 0.0)


def _tc_mm(n_rows, W, x=None, aggs=None, dins=None, brels=None, dout=None,
           b_out=None):
    nagg = 0 if x is not None else len(aggs)
    inputs, specs = [], []
    if x is not None:
        inputs.append(x)
        specs.append(pl.BlockSpec((_R, _D), lambda i: (i, 0)))
    else:
        for a, d in zip(aggs, dins):
            inputs.append(a)
            specs.append(pl.BlockSpec((4, _R, 32), lambda i: (0, i, 0)))
            inputs.append(d)
            specs.append(pl.BlockSpec((_R, 8), lambda i: (i, 0)))
        for br in brels:
            inputs.append(br.reshape(1, _D))
            specs.append(pl.BlockSpec((1, _D), lambda i: (0, 0)))
    if dout is not None:
        inputs.append(dout)
        specs.append(pl.BlockSpec((_R, 8), lambda i: (i, 0)))
    inputs.append(W)
    specs.append(pl.BlockSpec((_D, _D), lambda i: (0, 0)))
    if b_out is not None:
        inputs.append(b_out.reshape(1, _D))
        specs.append(pl.BlockSpec((1, _D), lambda i: (0, 0)))

    def body(*refs):
        o_ref = refs[-1]
        pos = 0
        if x is not None:
            h = refs[0][...]
            pos = 1
        else:
            h = jnp.zeros((_R, _D), jnp.float32)
            for k in range(nagg):
                a = refs[pos + 2 * k][...]
                d = refs[pos + 2 * k + 1][...][:, :1]
                br = refs[pos + 2 * nagg + k][...]
                cat = jnp.concatenate([a[0], a[1], a[2], a[3]], axis=1)
                h = h + jnp.maximum(cat * _inv_sqrt_deg(d) + br, 0.0)
            pos = 3 * nagg
        if dout is not None:
            h = h * _inv_sqrt_deg(refs[pos][...][:, :1])
            pos += 1
        z = jnp.dot(h, refs[pos][...], preferred_element_type=jnp.float32)
        pos += 1
        if b_out is not None:
            z = z + refs[pos][...]
        o_ref[...] = z

    return pl.pallas_call(
        body,
        grid=(n_rows // _R,),
        in_specs=specs,
        out_specs=pl.BlockSpec((_R, _D), lambda i: (i, 0)),
        out_shape=jax.ShapeDtypeStruct((n_rows, _D), jnp.float32),
    )(*inputs)


def kernel(x_gene, x_cell, x_gotem, src_g2c, dst_g2c, src_c2g, dst_c2g,
           src_g2go, dst_g2go, src_go2g, dst_go2g, params):
    xs = {"gene": x_gene, "cell": x_cell, "gotem": x_gotem}
    srcs = {"g2c": src_g2c, "c2g": src_c2g, "g2go": src_g2go,
            "go2g": src_go2g}
    dsts = {"g2c": dst_g2c, "c2g": dst_c2g, "g2go": dst_g2go,
            "go2g": dst_go2g}

    def pad(a, fill):
        return jnp.concatenate(
            [a, jnp.full((_EPAD - _E,), fill, jnp.int32)])

    gidx, dstt, sdeg = {}, {}, {}
    for r in _RELS:
        g = pad(srcs[r], 0) * 4
        gidx[r] = (g[None, :] +
                   jnp.arange(4, dtype=jnp.int32)[:, None]
                   ).reshape(4, _NSUB, 2, _KH, _CH)
        dstt[r] = pad(dsts[r], _NT[_DST_T[r]]).reshape(_NSUB, 2, _KH, _CH)
        sdeg[r] = pad(srcs[r], _NT[_SRC_T[r]]).reshape(_NSUB, _K, _CH)

    zeros32 = jnp.zeros((_ZBROWS, 32), jnp.float32)
    zeros8 = jnp.zeros((_ZROWS, 8), jnp.float32)
    ones8 = jnp.ones((_CH, 8), jnp.float32)

    deg_arrs, deg_npads = [], []
    for r in _RELS:
        deg_arrs += [sdeg[r], dstt[r].reshape(_NSUB, _K, _CH)]
        deg_npads += [_NPADS[_SRC_T[r]], _NPADS[_DST_T[r]]]
    degs = _degree_call(deg_arrs, deg_npads, ones8, zeros8)
    dout = {r: degs[2 * i] for i, r in enumerate(_RELS)}
    din = {r: degs[2 * i + 1] for i, r in enumerate(_RELS)}

    l0_rels = ("g2c", "c2g", "go2g")
    z0 = {r: _tc_mm(_NT[_SRC_T[r]], params["emb_gene"][r][0],
                    x=xs[_SRC_T[r]], dout=dout[r]) for r in l0_rels}

    def spec(r, z):
        return (z.reshape(-1, 32), gidx[r], dstt[r], _NPADS[_DST_T[r]])

    a0a = _sc_layer([spec(r, z0[r]) for r in ("c2g", "go2g")], zeros32)
    a0b_ = _sc_layer([spec("g2c", z0["g2c"])], zeros32)
    A0 = {"c2g": a0a[0], "go2g": a0a[1], "g2c": a0b_[0]}

    peg = params["emb_gene"]
    hspec = {
        "gene": ([A0["c2g"], A0["go2g"]], [din["c2g"], din["go2g"]],
                 [peg["c2g"][1], peg["go2g"][1]]),
        "cell": ([A0["g2c"]], [din["g2c"]], [peg["g2c"][1]]),
    }

    z0b = _tc_mm(_NG, params["emb_gotem"]["g2go"][0],
                 aggs=hspec["gene"][0], dins=hspec["gene"][1],
                 brels=hspec["gene"][2], dout=dout["g2go"])
    a0b = _sc_layer([spec("g2go", z0b)], zeros32)
    hspec["gotem"] = ([a0b[0]], [din["g2go"]],
                      [params["emb_gotem"]["g2go"][1]])

    for lname in ("conv1", "conv2", "conv3"):
        P = params[lname]
        zz = {r: _tc_mm(_NT[_SRC_T[r]], P[r][0],
                        aggs=hspec[_SRC_T[r]][0],
                        dins=hspec[_SRC_T[r]][1],
                        brels=hspec[_SRC_T[r]][2],
                        dout=dout[r]) for r in _RELS}
        aa = _sc_layer([spec(r, zz[r]) for r in ("c2g", "go2g")], zeros32)
        ab = _sc_layer([spec(r, zz[r]) for r in ("g2c", "g2go")], zeros32)
        A = {"c2g": aa[0], "go2g": aa[1], "g2c": ab[0], "g2go": ab[1]}
        hspec = {
            "gene": ([A["c2g"], A["go2g"]], [din["c2g"], din["go2g"]],
                     [P["c2g"][1], P["go2g"][1]]),
            "cell": ([A["g2c"]], [din["g2c"]], [P["g2c"][1]]),
            "gotem": ([A["g2go"]], [din["g2go"]], [P["g2go"][1]]),
        }

    return tuple(
        _tc_mm(_NT[t], params["dense"][t][0],
               aggs=hspec[t][0], dins=hspec[t][1], brels=hspec[t][2],
               b_out=params["dense"][t][1])
        for t in ("gene", "cell", "gotem"))


# speedup vs baseline: 2.8293x; 2.8293x over previous
import functools

import jax
import jax.numpy as jnp
from jax import lax
from jax.experimental import pallas as pl
from jax.experimental.pallas import tpu as pltpu
from jax.experimental.pallas import tpu_sc as plsc

_NG, _NC, _NGO, _E, _D = 50000, 50000, 10000, 150000, 128
_NSUB = 16
_NCORE = 2
_CH = 128
_K = 74
_EPAD = _NSUB * _K * _CH
_R = 1000

_RELS = ("g2c", "c2g", "g2go", "go2g")
_SRC_T = {"g2c": "gene", "c2g": "cell", "g2go": "gene", "go2g": "gotem"}
_DST_T = {"g2c": "cell", "c2g": "gene", "g2go": "gotem", "go2g": "gene"}
_NT = {"gene": _NG, "cell": _NC, "gotem": _NGO}


def _npad(n):
    return ((n + 1 + 127) // 128) * 128


_NPADS = {t: _npad(n) for t, n in _NT.items()}
_ZROWS = max(_NPADS.values()) // _NSUB
_ZBROWS = _ZROWS // 2

_MESH = functools.partial(
    plsc.VectorSubcoreMesh,
    core_axis_name="c", subcore_axis_name="s",
    num_cores=_NCORE, num_subcores=_NSUB,
)


def _degree_call(idx_arrs, npads, ones8, zeros8):
    n = len(idx_arrs)

    def body(*refs):
        idx_refs = refs[:n]
        ones_h, zeros_h = refs[n], refs[n + 1]
        out_refs = refs[n + 2: 2 * n + 2]
        idxbuf, onesbuf, acc = refs[2 * n + 2], refs[2 * n + 3], refs[2 * n + 4]
        cid = lax.axis_index("c")
        sid = lax.axis_index("s")
        pltpu.sync_copy(ones_h, onesbuf)
        for t in range(n):
            rows = npads[t] // _NSUB

            @pl.when(cid == (t % _NCORE))
            def _(t=t, rows=rows):
                pltpu.sync_copy(zeros_h.at[pl.ds(0, rows)],
                                acc.at[pl.ds(sid * rows, rows)])
                pltpu.sync_copy(idx_refs[t].at[sid], idxbuf)
                plsc.subcore_barrier()

                def step(j, carry):
                    pltpu.sync_copy(onesbuf, acc.at[idxbuf.at[j]], add=True)
                    return carry

                lax.fori_loop(0, _K, step, 0)
                plsc.subcore_barrier()
                pltpu.sync_copy(acc.at[pl.ds(sid * rows, rows)],
                                out_refs[t].at[pl.ds(sid * rows, rows)])
                plsc.subcore_barrier()

    out_type = [jax.ShapeDtypeStruct((p, 8), jnp.float32) for p in npads]
    f = pl.kernel(
        body,
        out_type=out_type,
        mesh=_MESH(),
        scratch_types=[
            pltpu.VMEM((_K, _CH), jnp.int32),
            pltpu.VMEM((_CH, 8), jnp.float32),
            pltpu.VMEM_SHARED((max(npads), 8), jnp.float32),
        ],
        compiler_params=pltpu.CompilerParams(use_tc_tiling_on_sc=False),
        name="rgcn_degrees",
    )
    return f(*idx_arrs, ones8, zeros8)


def _sc_layer(specs, zeros32):
    nrel = len(specs)
    npads = [s[3] for s in specs]

    def body(*refs):
        z_refs = refs[0:nrel]
        g_refs = refs[nrel:2 * nrel]
        d_refs = refs[2 * nrel:3 * nrel]
        zeros_h = refs[3 * nrel]
        out_refs = refs[3 * nrel + 1: 4 * nrel + 1]
        idxbuf, dstbuf, rb0, rb1, acc, g0, g1 = refs[4 * nrel + 1:]
        cid = lax.axis_index("c")
        sid = lax.axis_index("s")
        for t in range(nrel):
            rows = npads[t] // _NSUB
            pltpu.sync_copy(d_refs[t].at[sid], dstbuf)
            for gl in range(2):
                c = cid * 2 + gl
                for zo in range(0, rows, _ZBROWS):
                    zr = min(_ZBROWS, rows - zo)
                    pltpu.sync_copy(zeros_h.at[pl.ds(0, zr)],
                                    acc.at[pl.ds(sid * rows + zo, zr)])
                pltpu.sync_copy(g_refs[t].at[c, sid], idxbuf)
                plsc.subcore_barrier()

                z_ref = z_refs[t]
                pltpu.async_copy(z_ref.at[idxbuf.at[0]], rb0, g0)

                def round_(r, carry, t=t, z_ref=z_ref):
                    j = 2 * r
                    pltpu.async_copy(z_ref.at[idxbuf.at[j + 1]], rb1, g1)
                    pltpu.make_async_copy(z_ref.at[idxbuf.at[j]], rb0,
                                          g0).wait()
                    pltpu.sync_copy(rb0, acc.at[dstbuf.at[j]], add=True)

                    @pl.when(j + 2 < _K)
                    def _():
                        pltpu.async_copy(z_ref.at[idxbuf.at[j + 2]], rb0, g0)
                    pltpu.make_async_copy(z_ref.at[idxbuf.at[j + 1]], rb1,
                                          g1).wait()
                    pltpu.sync_copy(rb1, acc.at[dstbuf.at[j + 1]], add=True)
                    return carry

                lax.fori_loop(0, _K // 2, round_, 0)
                plsc.subcore_barrier()
                pltpu.sync_copy(acc.at[pl.ds(sid * rows, rows)],
                                out_refs[t].at[c, pl.ds(sid * rows, rows)])
                plsc.subcore_barrier()

    out_type = [jax.ShapeDtypeStruct((4, p, 32), jnp.float32) for p in npads]
    f = pl.kernel(
        body,
        out_type=out_type,
        mesh=_MESH(),
        scratch_types=[
            pltpu.VMEM((_K, _CH), jnp.int32),
            pltpu.VMEM((_K, _CH), jnp.int32),
            pltpu.VMEM((_CH, 32), jnp.float32),
            pltpu.VMEM((_CH, 32), jnp.float32),
            pltpu.VMEM_SHARED((max(npads), 32), jnp.float32),
            pltpu.SemaphoreType.DMA,
            pltpu.SemaphoreType.DMA,
        ],
        compiler_params=pltpu.CompilerParams(use_tc_tiling_on_sc=False),
        name="rgcn_sc_layer_%d" % nrel,
    )
    ins = []
    for grp in range(3):
        ins += [s[grp] for s in specs]
    outs = f(*ins, zeros32)
    return outs if isinstance(outs, (list, tuple)) else [outs]


def _inv_sqrt_deg(d):
    return jnp.where(d > 0, lax.rsqrt(jnp.maximum(d, 1.0)), 0.0)


def _tc_mm(n_rows, W, x=None, aggs=None, dins=None, brels=None, dout=None,
           b_out=None):
    nagg = 0 if x is not None else len(aggs)
    inputs, specs = [], []
    if x is not None:
        inputs.append(x)
        specs.append(pl.BlockSpec((_R, _D), lambda i: (i, 0)))
    else:
        for a, d in zip(aggs, dins):
            inputs.append(a)
            specs.append(pl.BlockSpec((4, _R, 32), lambda i: (0, i, 0)))
            inputs.append(d)
            specs.append(pl.BlockSpec((_R, 8), lambda i: (i, 0)))
        for br in brels:
            inputs.append(br.reshape(1, _D))
            specs.append(pl.BlockSpec((1, _D), lambda i: (0, 0)))
    if dout is not None:
        inputs.append(dout)
        specs.append(pl.BlockSpec((_R, 8), lambda i: (i, 0)))
    inputs.append(W)
    specs.append(pl.BlockSpec((_D, _D), lambda i: (0, 0)))
    if b_out is not None:
        inputs.append(b_out.reshape(1, _D))
        specs.append(pl.BlockSpec((1, _D), lambda i: (0, 0)))

    def body(*refs):
        o_ref = refs[-1]
        pos = 0
        if x is not None:
            h = refs[0][...]
            pos = 1
        else:
            h = jnp.zeros((_R, _D), jnp.float32)
            for k in range(nagg):
                a = refs[pos + 2 * k][...]
                d = refs[pos + 2 * k + 1][...][:, :1]
                br = refs[pos + 2 * nagg + k][...]
                cat = jnp.concatenate([a[0], a[1], a[2], a[3]], axis=1)
                h = h + jnp.maximum(cat * _inv_sqrt_deg(d) + br, 0.0)
            pos = 3 * nagg
        if dout is not None:
            h = h * _inv_sqrt_deg(refs[pos][...][:, :1])
            pos += 1
        z = jnp.dot(h, refs[pos][...], preferred_element_type=jnp.float32)
        pos += 1
        if b_out is not None:
            z = z + refs[pos][...]
        o_ref[...] = z

    return pl.pallas_call(
        body,
        grid=(n_rows // _R,),
        in_specs=specs,
        out_specs=pl.BlockSpec((_R, _D), lambda i: (i, 0)),
        out_shape=jax.ShapeDtypeStruct((n_rows, _D), jnp.float32),
    )(*inputs)


def kernel(x_gene, x_cell, x_gotem, src_g2c, dst_g2c, src_c2g, dst_c2g,
           src_g2go, dst_g2go, src_go2g, dst_go2g, params):
    xs = {"gene": x_gene, "cell": x_cell, "gotem": x_gotem}
    srcs = {"g2c": src_g2c, "c2g": src_c2g, "g2go": src_g2go,
            "go2g": src_go2g}
    dsts = {"g2c": dst_g2c, "c2g": dst_c2g, "g2go": dst_g2go,
            "go2g": dst_go2g}

    def pad(a, fill):
        return jnp.concatenate(
            [a, jnp.full((_EPAD - _E,), fill, jnp.int32)])

    gidx, dstt, sdeg = {}, {}, {}
    for r in _RELS:
        g = pad(srcs[r], 0) * 4
        gidx[r] = (g[None, :] +
                   jnp.arange(4, dtype=jnp.int32)[:, None]
                   ).reshape(4, _NSUB, _K, _CH)
        dstt[r] = pad(dsts[r], _NT[_DST_T[r]]).reshape(_NSUB, _K, _CH)
        sdeg[r] = pad(srcs[r], _NT[_SRC_T[r]]).reshape(_NSUB, _K, _CH)

    zeros32 = jnp.zeros((_ZBROWS, 32), jnp.float32)
    zeros8 = jnp.zeros((_ZROWS, 8), jnp.float32)
    ones8 = jnp.ones((_CH, 8), jnp.float32)

    deg_arrs, deg_npads = [], []
    for r in _RELS:
        deg_arrs += [sdeg[r], dstt[r]]
        deg_npads += [_NPADS[_SRC_T[r]], _NPADS[_DST_T[r]]]
    degs = _degree_call(deg_arrs, deg_npads, ones8, zeros8)
    dout = {r: degs[2 * i] for i, r in enumerate(_RELS)}
    din = {r: degs[2 * i + 1] for i, r in enumerate(_RELS)}

    l0_rels = ("g2c", "c2g", "go2g")
    z0 = {r: _tc_mm(_NT[_SRC_T[r]], params["emb_gene"][r][0],
                    x=xs[_SRC_T[r]], dout=dout[r]) for r in l0_rels}

    def spec(r, z):
        return (z.reshape(-1, 32), gidx[r], dstt[r], _NPADS[_DST_T[r]])

    a0a = _sc_layer([spec(r, z0[r]) for r in ("c2g", "go2g")], zeros32)
    a0b_ = _sc_layer([spec("g2c", z0["g2c"])], zeros32)
    A0 = {"c2g": a0a[0], "go2g": a0a[1], "g2c": a0b_[0]}

    peg = params["emb_gene"]
    hspec = {
        "gene": ([A0["c2g"], A0["go2g"]], [din["c2g"], din["go2g"]],
                 [peg["c2g"][1], peg["go2g"][1]]),
        "cell": ([A0["g2c"]], [din["g2c"]], [peg["g2c"][1]]),
    }

    z0b = _tc_mm(_NG, params["emb_gotem"]["g2go"][0],
                 aggs=hspec["gene"][0], dins=hspec["gene"][1],
                 brels=hspec["gene"][2], dout=dout["g2go"])
    a0b = _sc_layer([spec("g2go", z0b)], zeros32)
    hspec["gotem"] = ([a0b[0]], [din["g2go"]],
                      [params["emb_gotem"]["g2go"][1]])

    for lname in ("conv1", "conv2", "conv3"):
        P = params[lname]
        zz = {r: _tc_mm(_NT[_SRC_T[r]], P[r][0],
                        aggs=hspec[_SRC_T[r]][0],
                        dins=hspec[_SRC_T[r]][1],
                        brels=hspec[_SRC_T[r]][2],
                        dout=dout[r]) for r in _RELS}
        aa = _sc_layer([spec(r, zz[r]) for r in ("c2g", "go2g")], zeros32)
        ab = _sc_layer([spec(r, zz[r]) for r in ("g2c", "g2go")], zeros32)
        A = {"c2g": aa[0], "go2g": aa[1], "g2c": ab[0], "g2go": ab[1]}
        hspec = {
            "gene": ([A["c2g"], A["go2g"]], [din["c2g"], din["go2g"]],
                     [P["c2g"][1], P["go2g"][1]]),
            "cell": ([A["g2c"]], [din["g2c"]], [P["g2c"][1]]),
            "gotem": ([A["g2go"]], [din["g2go"]], [P["g2go"][1]]),
        }

    return tuple(
        _tc_mm(_NT[t], params["dense"][t][0],
               aggs=hspec[t][0], dins=hspec[t][1], brels=hspec[t][2],
               b_out=params["dense"][t][1])
        for t in ("gene", "cell", "gotem"))


# speedup vs baseline: 2.8932x; 1.0226x over previous
import functools

import jax
import jax.numpy as jnp
from jax import lax
from jax.experimental import pallas as pl
from jax.experimental.pallas import tpu as pltpu
from jax.experimental.pallas import tpu_sc as plsc

_NG, _NC, _NGO, _E, _D = 50000, 50000, 10000, 150000, 128
_NSUB = 16
_NCORE = 2
_CH = 128
_K = 74
_EPAD = _NSUB * _K * _CH
_R = 1000

_RELS = ("g2c", "c2g", "g2go", "go2g")
_SRC_T = {"g2c": "gene", "c2g": "cell", "g2go": "gene", "go2g": "gotem"}
_DST_T = {"g2c": "cell", "c2g": "gene", "g2go": "gotem", "go2g": "gene"}
_NT = {"gene": _NG, "cell": _NC, "gotem": _NGO}


def _npad(n):
    return ((n + 1 + 127) // 128) * 128


_NPADS = {t: _npad(n) for t, n in _NT.items()}
_ZROWS = max(_NPADS.values()) // _NSUB
_ZBROWS = _ZROWS // 2

_MESH = functools.partial(
    plsc.VectorSubcoreMesh,
    core_axis_name="c", subcore_axis_name="s",
    num_cores=_NCORE, num_subcores=_NSUB,
)


def _degree_call(idx_arrs, npads, ones8, zeros8):
    n = len(idx_arrs)

    def body(*refs):
        idx_refs = refs[:n]
        ones_h, zeros_h = refs[n], refs[n + 1]
        out_refs = refs[n + 2: 2 * n + 2]
        idxbuf, onesbuf, acc = refs[2 * n + 2], refs[2 * n + 3], refs[2 * n + 4]
        cid = lax.axis_index("c")
        sid = lax.axis_index("s")
        pltpu.sync_copy(ones_h, onesbuf)
        for t in range(n):
            rows = npads[t] // _NSUB

            @pl.when(cid == (t % _NCORE))
            def _(t=t, rows=rows):
                pltpu.sync_copy(zeros_h.at[pl.ds(0, rows)],
                                acc.at[pl.ds(sid * rows, rows)])
                pltpu.sync_copy(idx_refs[t].at[sid], idxbuf)
                plsc.subcore_barrier()

                def step(j, carry):
                    pltpu.sync_copy(onesbuf, acc.at[idxbuf.at[j]], add=True)
                    return carry

                lax.fori_loop(0, _K, step, 0)
                plsc.subcore_barrier()
                pltpu.sync_copy(acc.at[pl.ds(sid * rows, rows)],
                                out_refs[t].at[pl.ds(sid * rows, rows)])
                plsc.subcore_barrier()

    out_type = [jax.ShapeDtypeStruct((p, 8), jnp.float32) for p in npads]
    f = pl.kernel(
        body,
        out_type=out_type,
        mesh=_MESH(),
        scratch_types=[
            pltpu.VMEM((_K, _CH), jnp.int32),
            pltpu.VMEM((_CH, 8), jnp.float32),
            pltpu.VMEM_SHARED((max(npads), 8), jnp.float32),
        ],
        compiler_params=pltpu.CompilerParams(use_tc_tiling_on_sc=False),
        name="rgcn_degrees",
    )
    return f(*idx_arrs, ones8, zeros8)


def _sc_layer(specs, zeros32):
    nrel = len(specs)
    npads = [s[3] for s in specs]

    def body(*refs):
        z_refs = refs[0:nrel]
        g_refs = refs[nrel:2 * nrel]
        d_refs = refs[2 * nrel:3 * nrel]
        zeros_h = refs[3 * nrel]
        out_refs = refs[3 * nrel + 1: 4 * nrel + 1]
        idxbuf, dstbuf, rb0, rb1, acc, g0, g1 = refs[4 * nrel + 1:]
        cid = lax.axis_index("c")
        sid = lax.axis_index("s")
        for t in range(nrel):
            rows = npads[t] // _NSUB
            pltpu.sync_copy(d_refs[t].at[sid], dstbuf)
            for gl in range(2):
                c = cid * 2 + gl
                for zo in range(0, rows, _ZBROWS):
                    zr = min(_ZBROWS, rows - zo)
                    pltpu.sync_copy(zeros_h.at[pl.ds(0, zr)],
                                    acc.at[pl.ds(sid * rows + zo, zr)])
                pltpu.sync_copy(g_refs[t].at[c, sid], idxbuf)
                plsc.subcore_barrier()

                z_ref = z_refs[t]
                pltpu.async_copy(z_ref.at[idxbuf.at[0]], rb0, g0)

                def round_(r, carry, t=t, z_ref=z_ref):
                    j = 2 * r
                    pltpu.async_copy(z_ref.at[idxbuf.at[j + 1]], rb1, g1)
                    pltpu.make_async_copy(z_ref.at[idxbuf.at[j]], rb0,
                                          g0).wait()
                    pltpu.sync_copy(rb0, acc.at[dstbuf.at[j]], add=True)

                    @pl.when(j + 2 < _K)
                    def _():
                        pltpu.async_copy(z_ref.at[idxbuf.at[j + 2]], rb0, g0)
                    pltpu.make_async_copy(z_ref.at[idxbuf.at[j + 1]], rb1,
                                          g1).wait()
                    pltpu.sync_copy(rb1, acc.at[dstbuf.at[j + 1]], add=True)
                    return carry

                lax.fori_loop(0, _K // 2, round_, 0)
                plsc.subcore_barrier()
                pltpu.sync_copy(acc.at[pl.ds(sid * rows, rows)],
                                out_refs[t].at[c, pl.ds(sid * rows, rows)])
                plsc.subcore_barrier()

    out_type = [jax.ShapeDtypeStruct((4, p, 32), jnp.float32) for p in npads]
    f = pl.kernel(
        body,
        out_type=out_type,
        mesh=_MESH(),
        scratch_types=[
            pltpu.VMEM((_K, _CH), jnp.int32),
            pltpu.VMEM((_K, _CH), jnp.int32),
            pltpu.VMEM((_CH, 32), jnp.float32),
            pltpu.VMEM((_CH, 32), jnp.float32),
            pltpu.VMEM_SHARED((max(npads), 32), jnp.float32),
            pltpu.SemaphoreType.DMA,
            pltpu.SemaphoreType.DMA,
        ],
        compiler_params=pltpu.CompilerParams(use_tc_tiling_on_sc=False),
        name="rgcn_sc_layer_%d" % nrel,
    )
    ins = []
    for grp in range(3):
        ins += [s[grp] for s in specs]
    outs = f(*ins, zeros32)
    return outs if isinstance(outs, (list, tuple)) else [outs]


def _inv_sqrt_deg(d):
    return jnp.where(d > 0, lax.rsqrt(jnp.maximum(d, 1.0)), 0.0)


def _tc_mm(n_rows, W, x=None, aggs=None, dins=None, brels=None, dout=None,
           b_out=None):
    nagg = 0 if x is not None else len(aggs)
    inputs, specs = [], []
    if x is not None:
        inputs.append(x)
        specs.append(pl.BlockSpec((_R, _D), lambda i: (i, 0)))
    else:
        for a, d in zip(aggs, dins):
            inputs.append(a)
            specs.append(pl.BlockSpec((4, _R, 32), lambda i: (0, i, 0)))
            inputs.append(d)
            specs.append(pl.BlockSpec((_R, 8), lambda i: (i, 0)))
        for br in brels:
            inputs.append(br.reshape(1, _D))
            specs.append(pl.BlockSpec((1, _D), lambda i: (0, 0)))
    if dout is not None:
        inputs.append(dout)
        specs.append(pl.BlockSpec((_R, 8), lambda i: (i, 0)))
    inputs.append(W)
    specs.append(pl.BlockSpec((_D, _D), lambda i: (0, 0)))
    if b_out is not None:
        inputs.append(b_out.reshape(1, _D))
        specs.append(pl.BlockSpec((1, _D), lambda i: (0, 0)))

    def body(*refs):
        o_ref = refs[-1]
        pos = 0
        if x is not None:
            h = refs[0][...]
            pos = 1
        else:
            h = jnp.zeros((_R, _D), jnp.float32)
            for k in range(nagg):
                a = refs[pos + 2 * k][...]
                d = refs[pos + 2 * k + 1][...][:, :1]
                br = refs[pos + 2 * nagg + k][...]
                cat = jnp.concatenate([a[0], a[1], a[2], a[3]], axis=1)
                h = h + jnp.maximum(cat * _inv_sqrt_deg(d) + br, 0.0)
            pos = 3 * nagg
        if dout is not None:
            h = h * _inv_sqrt_deg(refs[pos][...][:, :1])
            pos += 1
        z = jnp.dot(h, refs[pos][...], preferred_element_type=jnp.float32)
        pos += 1
        if b_out is not None:
            z = z + refs[pos][...]
        o_ref[...] = z

    return pl.pallas_call(
        body,
        grid=(n_rows // _R,),
        in_specs=specs,
        out_specs=pl.BlockSpec((_R, _D), lambda i: (i, 0)),
        out_shape=jax.ShapeDtypeStruct((n_rows, _D), jnp.float32),
    )(*inputs)


def kernel(x_gene, x_cell, x_gotem, src_g2c, dst_g2c, src_c2g, dst_c2g,
           src_g2go, dst_g2go, src_go2g, dst_go2g, params):
    xs = {"gene": x_gene, "cell": x_cell, "gotem": x_gotem}
    srcs = {"g2c": src_g2c, "c2g": src_c2g, "g2go": src_g2go,
            "go2g": src_go2g}
    dsts = {"g2c": dst_g2c, "c2g": dst_c2g, "g2go": dst_g2go,
            "go2g": dst_go2g}

    def pad(a, fill):
        return jnp.concatenate(
            [a, jnp.full((_EPAD - _E,), fill, jnp.int32)])

    gidx, dstt, sdeg = {}, {}, {}
    for r in _RELS:
        g = pad(srcs[r], 0) * 4
        gidx[r] = (g[None, :] +
                   jnp.arange(4, dtype=jnp.int32)[:, None]
                   ).reshape(4, _NSUB, _K, _CH)
        dstt[r] = pad(dsts[r], _NT[_DST_T[r]]).reshape(_NSUB, _K, _CH)
        sdeg[r] = pad(srcs[r], _NT[_SRC_T[r]]).reshape(_NSUB, _K, _CH)

    zeros32 = jnp.zeros((_ZBROWS, 32), jnp.float32)
    zeros8 = jnp.zeros((_ZROWS, 8), jnp.float32)
    ones8 = jnp.ones((_CH, 8), jnp.float32)

    deg_arrs, deg_npads = [], []
    for r in _RELS:
        deg_arrs += [sdeg[r], dstt[r]]
        deg_npads += [_NPADS[_SRC_T[r]], _NPADS[_DST_T[r]]]
    degs = _degree_call(deg_arrs, deg_npads, ones8, zeros8)
    dout = {r: degs[2 * i] for i, r in enumerate(_RELS)}
    din = {r: degs[2 * i + 1] for i, r in enumerate(_RELS)}

    l0_rels = ("g2c", "c2g", "go2g")
    z0 = {r: _tc_mm(_NT[_SRC_T[r]], params["emb_gene"][r][0],
                    x=xs[_SRC_T[r]], dout=dout[r]) for r in l0_rels}

    def spec(r, z):
        return (z.reshape(-1, 32), gidx[r], dstt[r], _NPADS[_DST_T[r]])

    a0a = _sc_layer([spec(r, z0[r]) for r in ("c2g", "go2g")], zeros32)
    a0b_ = _sc_layer([spec("g2c", z0["g2c"])], zeros32)
    A0 = {"c2g": a0a[0], "go2g": a0a[1], "g2c": a0b_[0]}

    peg = params["emb_gene"]
    hspec = {
        "gene": ([A0["c2g"], A0["go2g"]], [din["c2g"], din["go2g"]],
                 [peg["c2g"][1], peg["go2g"][1]]),
        "cell": ([A0["g2c"]], [din["g2c"]], [peg["g2c"][1]]),
    }

    z0b = _tc_mm(_NG, params["emb_gotem"]["g2go"][0],
                 aggs=hspec["gene"][0], dins=hspec["gene"][1],
                 brels=hspec["gene"][2], dout=dout["g2go"])
    a0b = _sc_layer([spec("g2go", z0b)], zeros32)
    hspec["gotem"] = ([a0b[0]], [din["g2go"]],
                      [params["emb_gotem"]["g2go"][1]])

    for lname in ("conv1", "conv2", "conv3"):
        P = params[lname]
        zz = {r: _tc_mm(_NT[_SRC_T[r]], P[r][0],
                        aggs=hspec[_SRC_T[r]][0],
                        dins=hspec[_SRC_T[r]][1],
                        brels=hspec[_SRC_T[r]][2],
                        dout=dout[r]) for r in _RELS}
        aa = _sc_layer([spec(r, zz[r]) for r in ("c2g", "go2g")], zeros32)
        ab1 = _sc_layer([spec("g2c", zz["g2c"])], zeros32)
        ab2 = _sc_layer([spec("g2go", zz["g2go"])], zeros32)
        A = {"c2g": aa[0], "go2g": aa[1], "g2c": ab1[0], "g2go": ab2[0]}
        hspec = {
            "gene": ([A["c2g"], A["go2g"]], [din["c2g"], din["go2g"]],
                     [P["c2g"][1], P["go2g"][1]]),
            "cell": ([A["g2c"]], [din["g2c"]], [P["g2c"][1]]),
            "gotem": ([A["g2go"]], [din["g2go"]], [P["g2go"][1]]),
        }

    return tuple(
        _tc_mm(_NT[t], params["dense"][t][0],
               aggs=hspec[t][0], dins=hspec[t][1], brels=hspec[t][2],
               b_out=params["dense"][t][1])
        for t in ("gene", "cell", "gotem"))


# speedup vs baseline: 3.7086x; 1.2818x over previous
import functools

import jax
import jax.numpy as jnp
from jax import lax
from jax.experimental import pallas as pl
from jax.experimental.pallas import tpu as pltpu
from jax.experimental.pallas import tpu_sc as plsc

_NG, _NC, _NGO, _E, _D = 50000, 50000, 10000, 150000, 128
_NSUB = 16
_NCORE = 2
_CH = 128
_K = 74
_EPAD = _NSUB * _K * _CH
_R = 2000

_RELS = ("g2c", "c2g", "g2go", "go2g")
_SRC_T = {"g2c": "gene", "c2g": "cell", "g2go": "gene", "go2g": "gotem"}
_DST_T = {"g2c": "cell", "c2g": "gene", "g2go": "gotem", "go2g": "gene"}
_NT = {"gene": _NG, "cell": _NC, "gotem": _NGO}


def _npad(n):
    return ((n + 1 + 127) // 128) * 128


_NPADS = {t: _npad(n) for t, n in _NT.items()}
_ZROWS = max(_NPADS.values()) // _NSUB
_ZBROWS = _ZROWS // 2

_MESH = functools.partial(
    plsc.VectorSubcoreMesh,
    core_axis_name="c", subcore_axis_name="s",
    num_cores=_NCORE, num_subcores=_NSUB,
)


def _degree_call(idx_arrs, npads, ones8, zeros8):
    n = len(idx_arrs)

    def body(*refs):
        idx_refs = refs[:n]
        ones_h, zeros_h = refs[n], refs[n + 1]
        out_refs = refs[n + 2: 2 * n + 2]
        idxbuf, onesbuf, acc = refs[2 * n + 2], refs[2 * n + 3], refs[2 * n + 4]
        cid = lax.axis_index("c")
        sid = lax.axis_index("s")
        pltpu.sync_copy(ones_h, onesbuf)
        for t in range(n):
            rows = npads[t] // _NSUB

            @pl.when(cid == (t % _NCORE))
            def _(t=t, rows=rows):
                pltpu.sync_copy(zeros_h.at[pl.ds(0, rows)],
                                acc.at[pl.ds(sid * rows, rows)])
                pltpu.sync_copy(idx_refs[t].at[sid], idxbuf)
                plsc.subcore_barrier()

                def step(j, carry):
                    pltpu.sync_copy(onesbuf, acc.at[idxbuf.at[j]], add=True)
                    return carry

                lax.fori_loop(0, _K, step, 0)
                plsc.subcore_barrier()
                pltpu.sync_copy(acc.at[pl.ds(sid * rows, rows)],
                                out_refs[t].at[pl.ds(sid * rows, rows)])
                plsc.subcore_barrier()

    out_type = [jax.ShapeDtypeStruct((p, 8), jnp.float32) for p in npads]
    f = pl.kernel(
        body,
        out_type=out_type,
        mesh=_MESH(),
        scratch_types=[
            pltpu.VMEM((_K, _CH), jnp.int32),
            pltpu.VMEM((_CH, 8), jnp.float32),
            pltpu.VMEM_SHARED((max(npads), 8), jnp.float32),
        ],
        compiler_params=pltpu.CompilerParams(use_tc_tiling_on_sc=False),
        name="rgcn_degrees",
    )
    return f(*idx_arrs, ones8, zeros8)


def _sc_layer(specs, zeros32):
    nrel = len(specs)
    npads = [s[3] for s in specs]

    def body(*refs):
        z_refs = refs[0:nrel]
        g_refs = refs[nrel:2 * nrel]
        d_refs = refs[2 * nrel:3 * nrel]
        zeros_h = refs[3 * nrel]
        out_refs = refs[3 * nrel + 1: 4 * nrel + 1]
        idxbuf, dstbuf, rb0, rb1, acc, g0, g1 = refs[4 * nrel + 1:]
        cid = lax.axis_index("c")
        sid = lax.axis_index("s")
        c = cid
        for t in range(nrel):
            rows = npads[t] // _NSUB
            pltpu.sync_copy(d_refs[t].at[sid], dstbuf)
            if True:
                for zo in range(0, rows, _ZBROWS):
                    zr = min(_ZBROWS, rows - zo)
                    pltpu.sync_copy(zeros_h.at[pl.ds(0, zr)],
                                    acc.at[pl.ds(sid * rows + zo, zr)])
                pltpu.sync_copy(g_refs[t].at[c, sid], idxbuf)
                plsc.subcore_barrier()

                z_ref = z_refs[t]
                pltpu.async_copy(z_ref.at[idxbuf.at[0]], rb0, g0)

                def round_(r, carry, t=t, z_ref=z_ref):
                    j = 2 * r
                    pltpu.async_copy(z_ref.at[idxbuf.at[j + 1]], rb1, g1)
                    pltpu.make_async_copy(z_ref.at[idxbuf.at[j]], rb0,
                                          g0).wait()
                    pltpu.sync_copy(rb0, acc.at[dstbuf.at[j]], add=True)

                    @pl.when(j + 2 < _K)
                    def _():
                        pltpu.async_copy(z_ref.at[idxbuf.at[j + 2]], rb0, g0)
                    pltpu.make_async_copy(z_ref.at[idxbuf.at[j + 1]], rb1,
                                          g1).wait()
                    pltpu.sync_copy(rb1, acc.at[dstbuf.at[j + 1]], add=True)
                    return carry

                lax.fori_loop(0, _K // 2, round_, 0)
                plsc.subcore_barrier()
                pltpu.sync_copy(acc.at[pl.ds(sid * rows, rows)],
                                out_refs[t].at[c, pl.ds(sid * rows, rows)])
                plsc.subcore_barrier()

    out_type = [jax.ShapeDtypeStruct((2, p, 64), jnp.bfloat16) for p in npads]
    f = pl.kernel(
        body,
        out_type=out_type,
        mesh=_MESH(),
        scratch_types=[
            pltpu.VMEM((_K, _CH), jnp.int32),
            pltpu.VMEM((_K, _CH), jnp.int32),
            pltpu.VMEM((_CH, 64), jnp.bfloat16),
            pltpu.VMEM((_CH, 64), jnp.bfloat16),
            pltpu.VMEM_SHARED((max(npads), 64), jnp.bfloat16),
            pltpu.SemaphoreType.DMA,
            pltpu.SemaphoreType.DMA,
        ],
        compiler_params=pltpu.CompilerParams(use_tc_tiling_on_sc=False),
        name="rgcn_sc_layer_%d" % nrel,
    )
    ins = []
    for grp in range(3):
        ins += [s[grp] for s in specs]
    outs = f(*ins, zeros32)
    return outs if isinstance(outs, (list, tuple)) else [outs]


def _inv_sqrt_deg(d):
    return jnp.where(d > 0, lax.rsqrt(jnp.maximum(d, 1.0)), 0.0)


def _tc_mm(n_rows, W, x=None, aggs=None, dins=None, brels=None, dout=None,
           b_out=None, out_bf16=False):
    nagg = 0 if x is not None else len(aggs)
    inputs, specs = [], []
    if x is not None:
        inputs.append(x)
        specs.append(pl.BlockSpec((_R, _D), lambda i: (i, 0)))
    else:
        for a, d in zip(aggs, dins):
            inputs.append(a)
            specs.append(pl.BlockSpec((2, _R, 64), lambda i: (0, i, 0)))
            inputs.append(d)
            specs.append(pl.BlockSpec((_R, 8), lambda i: (i, 0)))
        for br in brels:
            inputs.append(br.reshape(1, _D))
            specs.append(pl.BlockSpec((1, _D), lambda i: (0, 0)))
    if dout is not None:
        inputs.append(dout)
        specs.append(pl.BlockSpec((_R, 8), lambda i: (i, 0)))
    inputs.append(W)
    specs.append(pl.BlockSpec((_D, _D), lambda i: (0, 0)))
    if b_out is not None:
        inputs.append(b_out.reshape(1, _D))
        specs.append(pl.BlockSpec((1, _D), lambda i: (0, 0)))

    def body(*refs):
        o_ref = refs[-1]
        pos = 0
        if x is not None:
            h = refs[0][...]
            pos = 1
        else:
            h = jnp.zeros((_R, _D), jnp.float32)
            for k in range(nagg):
                a = refs[pos + 2 * k][...]
                d = refs[pos + 2 * k + 1][...][:, :1]
                br = refs[pos + 2 * nagg + k][...]
                cat = jnp.concatenate([a[0], a[1]],
                                      axis=1).astype(jnp.float32)
                h = h + jnp.maximum(cat * _inv_sqrt_deg(d) + br, 0.0)
            pos = 3 * nagg
        if dout is not None:
            h = h * _inv_sqrt_deg(refs[pos][...][:, :1])
            pos += 1
        z = jnp.dot(h, refs[pos][...], preferred_element_type=jnp.float32)
        pos += 1
        if b_out is not None:
            z = z + refs[pos][...]
        o_ref[...] = z.astype(jnp.bfloat16) if out_bf16 else z

    return pl.pallas_call(
        body,
        grid=(n_rows // _R,),
        in_specs=specs,
        out_specs=pl.BlockSpec((_R, _D), lambda i: (i, 0)),
        out_shape=jax.ShapeDtypeStruct(
            (n_rows, _D), jnp.bfloat16 if out_bf16 else jnp.float32),
    )(*inputs)


def kernel(x_gene, x_cell, x_gotem, src_g2c, dst_g2c, src_c2g, dst_c2g,
           src_g2go, dst_g2go, src_go2g, dst_go2g, params):
    xs = {"gene": x_gene, "cell": x_cell, "gotem": x_gotem}
    srcs = {"g2c": src_g2c, "c2g": src_c2g, "g2go": src_g2go,
            "go2g": src_go2g}
    dsts = {"g2c": dst_g2c, "c2g": dst_c2g, "g2go": dst_g2go,
            "go2g": dst_go2g}

    def pad(a, fill):
        return jnp.concatenate(
            [a, jnp.full((_EPAD - _E,), fill, jnp.int32)])

    gidx, dstt, sdeg = {}, {}, {}
    for r in _RELS:
        g = pad(srcs[r], 0) * 2
        gidx[r] = (g[None, :] +
                   jnp.arange(2, dtype=jnp.int32)[:, None]
                   ).reshape(2, _NSUB, _K, _CH)
        dstt[r] = pad(dsts[r], _NT[_DST_T[r]]).reshape(_NSUB, _K, _CH)
        sdeg[r] = pad(srcs[r], _NT[_SRC_T[r]]).reshape(_NSUB, _K, _CH)

    zeros32 = jnp.zeros((_ZBROWS, 64), jnp.bfloat16)
    zeros8 = jnp.zeros((_ZROWS, 8), jnp.float32)
    ones8 = jnp.ones((_CH, 8), jnp.float32)

    deg_arrs, deg_npads = [], []
    for r in _RELS:
        deg_arrs += [sdeg[r], dstt[r]]
        deg_npads += [_NPADS[_SRC_T[r]], _NPADS[_DST_T[r]]]
    degs = _degree_call(deg_arrs, deg_npads, ones8, zeros8)
    dout = {r: degs[2 * i] for i, r in enumerate(_RELS)}
    din = {r: degs[2 * i + 1] for i, r in enumerate(_RELS)}

    l0_rels = ("g2c", "c2g", "go2g")
    z0 = {r: _tc_mm(_NT[_SRC_T[r]], params["emb_gene"][r][0],
                    x=xs[_SRC_T[r]], dout=dout[r], out_bf16=True)
          for r in l0_rels}

    def spec(r, z):
        return (z.reshape(-1, 64), gidx[r], dstt[r], _NPADS[_DST_T[r]])

    a0a = _sc_layer([spec(r, z0[r]) for r in ("c2g", "go2g")], zeros32)
    a0b_ = _sc_layer([spec("g2c", z0["g2c"])], zeros32)
    A0 = {"c2g": a0a[0], "go2g": a0a[1], "g2c": a0b_[0]}

    peg = params["emb_gene"]
    hspec = {
        "gene": ([A0["c2g"], A0["go2g"]], [din["c2g"], din["go2g"]],
                 [peg["c2g"][1], peg["go2g"][1]]),
        "cell": ([A0["g2c"]], [din["g2c"]], [peg["g2c"][1]]),
    }

    z0b = _tc_mm(_NG, params["emb_gotem"]["g2go"][0],
                 aggs=hspec["gene"][0], dins=hspec["gene"][1],
                 brels=hspec["gene"][2], dout=dout["g2go"], out_bf16=True)
    a0b = _sc_layer([spec("g2go", z0b)], zeros32)
    hspec["gotem"] = ([a0b[0]], [din["g2go"]],
                      [params["emb_gotem"]["g2go"][1]])

    for lname in ("conv1", "conv2", "conv3"):
        P = params[lname]
        zz = {r: _tc_mm(_NT[_SRC_T[r]], P[r][0],
                        aggs=hspec[_SRC_T[r]][0],
                        dins=hspec[_SRC_T[r]][1],
                        brels=hspec[_SRC_T[r]][2],
                        dout=dout[r], out_bf16=True) for r in _RELS}
        aa = _sc_layer([spec(r, zz[r]) for r in ("c2g", "go2g")], zeros32)
        ab1 = _sc_layer([spec("g2c", zz["g2c"])], zeros32)
        ab2 = _sc_layer([spec("g2go", zz["g2go"])], zeros32)
        A = {"c2g": aa[0], "go2g": aa[1], "g2c": ab1[0], "g2go": ab2[0]}
        hspec = {
            "gene": ([A["c2g"], A["go2g"]], [din["c2g"], din["go2g"]],
                     [P["c2g"][1], P["go2g"][1]]),
            "cell": ([A["g2c"]], [din["g2c"]], [P["g2c"][1]]),
            "gotem": ([A["g2go"]], [din["g2go"]], [P["g2go"][1]]),
        }

    return tuple(
        _tc_mm(_NT[t], params["dense"][t][0],
               aggs=hspec[t][0], dins=hspec[t][1], brels=hspec[t][2],
               b_out=params["dense"][t][1])
        for t in ("gene", "cell", "gotem"))


# speedup vs baseline: 3.7955x; 1.0234x over previous
import functools

import jax
import jax.numpy as jnp
from jax import lax
from jax.experimental import pallas as pl
from jax.experimental.pallas import tpu as pltpu
from jax.experimental.pallas import tpu_sc as plsc

_NG, _NC, _NGO, _E, _D = 50000, 50000, 10000, 150000, 128
_NSUB = 16
_NCORE = 2
_CH = 128
_K = 74
_EPAD = _NSUB * _K * _CH
_R = 2000

_RELS = ("g2c", "c2g", "g2go", "go2g")
_SRC_T = {"g2c": "gene", "c2g": "cell", "g2go": "gene", "go2g": "gotem"}
_DST_T = {"g2c": "cell", "c2g": "gene", "g2go": "gotem", "go2g": "gene"}
_NT = {"gene": _NG, "cell": _NC, "gotem": _NGO}


def _npad(n):
    return ((n + 1 + 127) // 128) * 128


_NPADS = {t: _npad(n) for t, n in _NT.items()}
_ZROWS = max(_NPADS.values()) // _NSUB
_ZBROWS = _ZROWS // 2

_MESH = functools.partial(
    plsc.VectorSubcoreMesh,
    core_axis_name="c", subcore_axis_name="s",
    num_cores=_NCORE, num_subcores=_NSUB,
)


def _degree_call(idx_arrs, npads, ones8, zeros8):
    n = len(idx_arrs)

    def body(*refs):
        idx_refs = refs[:n]
        ones_h, zeros_h = refs[n], refs[n + 1]
        out_refs = refs[n + 2: 2 * n + 2]
        idxbuf, onesbuf, acc = refs[2 * n + 2], refs[2 * n + 3], refs[2 * n + 4]
        cid = lax.axis_index("c")
        sid = lax.axis_index("s")
        pltpu.sync_copy(ones_h, onesbuf)
        for t in range(n):
            rows = npads[t] // _NSUB

            @pl.when(cid == (t % _NCORE))
            def _(t=t, rows=rows):
                pltpu.sync_copy(zeros_h.at[pl.ds(0, rows)],
                                acc.at[pl.ds(sid * rows, rows)])
                pltpu.sync_copy(idx_refs[t].at[sid], idxbuf)
                plsc.subcore_barrier()

                def step(j, carry):
                    pltpu.sync_copy(onesbuf, acc.at[idxbuf.at[j]], add=True)
                    return carry

                lax.fori_loop(0, _K, step, 0)
                plsc.subcore_barrier()
                pltpu.sync_copy(acc.at[pl.ds(sid * rows, rows)],
                                out_refs[t].at[pl.ds(sid * rows, rows)])
                plsc.subcore_barrier()

    out_type = [jax.ShapeDtypeStruct((p, 8), jnp.float32) for p in npads]
    f = pl.kernel(
        body,
        out_type=out_type,
        mesh=_MESH(),
        scratch_types=[
            pltpu.VMEM((_K, _CH), jnp.int32),
            pltpu.VMEM((_CH, 8), jnp.float32),
            pltpu.VMEM_SHARED((max(npads), 8), jnp.float32),
        ],
        compiler_params=pltpu.CompilerParams(use_tc_tiling_on_sc=False),
        name="rgcn_degrees",
    )
    return f(*idx_arrs, ones8, zeros8)


def _sc_layer(specs, zeros32):
    nrel = len(specs)
    npads = [s[3] for s in specs]

    def body(*refs):
        z_refs = refs[0:nrel]
        g_refs = refs[nrel:2 * nrel]
        d_refs = refs[2 * nrel:3 * nrel]
        zeros_h = refs[3 * nrel]
        out_refs = refs[3 * nrel + 1: 4 * nrel + 1]
        idxbuf, dstbuf, rb0, rb1, acc, g0, g1 = refs[4 * nrel + 1:]
        cid = lax.axis_index("c")
        sid = lax.axis_index("s")
        c = cid
        for t in range(nrel):
            rows = npads[t] // _NSUB
            pltpu.sync_copy(d_refs[t].at[sid], dstbuf)
            if True:
                for zo in range(0, rows, _ZBROWS):
                    zr = min(_ZBROWS, rows - zo)
                    pltpu.sync_copy(zeros_h.at[pl.ds(0, zr)],
                                    acc.at[pl.ds(sid * rows + zo, zr)])
                pltpu.sync_copy(g_refs[t].at[c, sid], idxbuf)
                plsc.subcore_barrier()

                z_ref = z_refs[t]
                pltpu.async_copy(z_ref.at[idxbuf.at[0]], rb0, g0)

                def round_(r, carry, t=t, z_ref=z_ref):
                    j = 2 * r
                    pltpu.async_copy(z_ref.at[idxbuf.at[j + 1]], rb1, g1)
                    pltpu.make_async_copy(z_ref.at[idxbuf.at[j]], rb0,
                                          g0).wait()
                    pltpu.sync_copy(rb0, acc.at[dstbuf.at[j]], add=True)

                    @pl.when(j + 2 < _K)
                    def _():
                        pltpu.async_copy(z_ref.at[idxbuf.at[j + 2]], rb0, g0)
                    pltpu.make_async_copy(z_ref.at[idxbuf.at[j + 1]], rb1,
                                          g1).wait()
                    pltpu.sync_copy(rb1, acc.at[dstbuf.at[j + 1]], add=True)
                    return carry

                lax.fori_loop(0, _K // 2, round_, 0)
                plsc.subcore_barrier()
                pltpu.sync_copy(acc.at[pl.ds(sid * rows, rows)],
                                out_refs[t].at[c, pl.ds(sid * rows, rows)])
                plsc.subcore_barrier()

    out_type = [jax.ShapeDtypeStruct((2, p, 64), jnp.bfloat16) for p in npads]
    f = pl.kernel(
        body,
        out_type=out_type,
        mesh=_MESH(),
        scratch_types=[
            pltpu.VMEM((_K, _CH), jnp.int32),
            pltpu.VMEM((_K, _CH), jnp.int32),
            pltpu.VMEM((_CH, 64), jnp.bfloat16),
            pltpu.VMEM((_CH, 64), jnp.bfloat16),
            pltpu.VMEM_SHARED((max(npads), 64), jnp.bfloat16),
            pltpu.SemaphoreType.DMA,
            pltpu.SemaphoreType.DMA,
        ],
        compiler_params=pltpu.CompilerParams(use_tc_tiling_on_sc=False),
        name="rgcn_sc_layer_%d" % nrel,
    )
    ins = []
    for grp in range(3):
        ins += [s[grp] for s in specs]
    outs = f(*ins, zeros32)
    return outs if isinstance(outs, (list, tuple)) else [outs]


def _inv_sqrt_deg(d):
    return jnp.where(d > 0, lax.rsqrt(jnp.maximum(d, 1.0)), 0.0)


def _tc_mm(n_rows, W, x=None, aggs=None, dins=None, brels=None, dout=None,
           b_out=None, out_bf16=False):
    nagg = 0 if x is not None else len(aggs)
    inputs, specs = [], []
    if x is not None:
        inputs.append(x)
        specs.append(pl.BlockSpec((_R, _D), lambda i: (i, 0)))
    else:
        for a, d in zip(aggs, dins):
            inputs.append(a)
            specs.append(pl.BlockSpec((2, _R, 64), lambda i: (0, i, 0)))
            inputs.append(d)
            specs.append(pl.BlockSpec((_R, 8), lambda i: (i, 0)))
        for br in brels:
            inputs.append(br.reshape(1, _D))
            specs.append(pl.BlockSpec((1, _D), lambda i: (0, 0)))
    if dout is not None:
        inputs.append(dout)
        specs.append(pl.BlockSpec((_R, 8), lambda i: (i, 0)))
    inputs.append(W)
    specs.append(pl.BlockSpec((_D, _D), lambda i: (0, 0)))
    if b_out is not None:
        inputs.append(b_out.reshape(1, _D))
        specs.append(pl.BlockSpec((1, _D), lambda i: (0, 0)))

    def body(*refs):
        o_ref = refs[-1]
        pos = 0
        if x is not None:
            h = refs[0][...]
            pos = 1
        else:
            h = jnp.zeros((_R, _D), jnp.float32)
            for k in range(nagg):
                a = refs[pos + 2 * k][...]
                d = refs[pos + 2 * k + 1][...][:, :1]
                br = refs[pos + 2 * nagg + k][...]
                cat = jnp.concatenate([a[0], a[1]],
                                      axis=1).astype(jnp.float32)
                h = h + jnp.maximum(cat * _inv_sqrt_deg(d) + br, 0.0)
            pos = 3 * nagg
        if dout is not None:
            h = h * _inv_sqrt_deg(refs[pos][...][:, :1])
            pos += 1
        z = jnp.dot(h, refs[pos][...], preferred_element_type=jnp.float32)
        pos += 1
        if b_out is not None:
            z = z + refs[pos][...]
        o_ref[...] = z.astype(jnp.bfloat16) if out_bf16 else z

    return pl.pallas_call(
        body,
        grid=(n_rows // _R,),
        in_specs=specs,
        out_specs=pl.BlockSpec((_R, _D), lambda i: (i, 0)),
        out_shape=jax.ShapeDtypeStruct(
            (n_rows, _D), jnp.bfloat16 if out_bf16 else jnp.float32),
    )(*inputs)


def kernel(x_gene, x_cell, x_gotem, src_g2c, dst_g2c, src_c2g, dst_c2g,
           src_g2go, dst_g2go, src_go2g, dst_go2g, params):
    xs = {"gene": x_gene, "cell": x_cell, "gotem": x_gotem}
    srcs = {"g2c": src_g2c, "c2g": src_c2g, "g2go": src_g2go,
            "go2g": src_go2g}
    dsts = {"g2c": dst_g2c, "c2g": dst_c2g, "g2go": dst_g2go,
            "go2g": dst_go2g}

    def pad(a, fill):
        return jnp.concatenate(
            [a, jnp.full((_EPAD - _E,), fill, jnp.int32)])

    gidx, dstt, sdeg = {}, {}, {}
    for r in _RELS:
        g = pad(srcs[r], 0) * 2
        gidx[r] = (g[None, :] +
                   jnp.arange(2, dtype=jnp.int32)[:, None]
                   ).reshape(2, _NSUB, _K, _CH)
        dstt[r] = pad(dsts[r], _NT[_DST_T[r]]).reshape(_NSUB, _K, _CH)
        sdeg[r] = pad(srcs[r], _NT[_SRC_T[r]]).reshape(_NSUB, _K, _CH)

    zeros32 = jnp.zeros((_ZBROWS, 64), jnp.bfloat16)
    zeros8 = jnp.zeros((_ZROWS, 8), jnp.float32)
    ones8 = jnp.ones((_CH, 8), jnp.float32)

    deg_arrs, deg_npads = [], []
    for r in _RELS:
        deg_arrs += [sdeg[r], dstt[r]]
        deg_npads += [_NPADS[_SRC_T[r]], _NPADS[_DST_T[r]]]
    degs = _degree_call(deg_arrs, deg_npads, ones8, zeros8)
    dout = {r: degs[2 * i] for i, r in enumerate(_RELS)}
    din = {r: degs[2 * i + 1] for i, r in enumerate(_RELS)}

    l0_rels = ("g2c", "c2g", "go2g")
    z0 = {r: _tc_mm(_NT[_SRC_T[r]], params["emb_gene"][r][0],
                    x=xs[_SRC_T[r]], dout=dout[r], out_bf16=True)
          for r in l0_rels}

    def spec(r, z):
        return (z.reshape(-1, 64), gidx[r], dstt[r], _NPADS[_DST_T[r]])

    a0a1 = _sc_layer([spec("c2g", z0["c2g"])], zeros32)
    a0a2 = _sc_layer([spec("go2g", z0["go2g"])], zeros32)
    a0b_ = _sc_layer([spec("g2c", z0["g2c"])], zeros32)
    A0 = {"c2g": a0a1[0], "go2g": a0a2[0], "g2c": a0b_[0]}

    peg = params["emb_gene"]
    hspec = {
        "gene": ([A0["c2g"], A0["go2g"]], [din["c2g"], din["go2g"]],
                 [peg["c2g"][1], peg["go2g"][1]]),
        "cell": ([A0["g2c"]], [din["g2c"]], [peg["g2c"][1]]),
    }

    z0b = _tc_mm(_NG, params["emb_gotem"]["g2go"][0],
                 aggs=hspec["gene"][0], dins=hspec["gene"][1],
                 brels=hspec["gene"][2], dout=dout["g2go"], out_bf16=True)
    a0b = _sc_layer([spec("g2go", z0b)], zeros32)
    hspec["gotem"] = ([a0b[0]], [din["g2go"]],
                      [params["emb_gotem"]["g2go"][1]])

    for lname in ("conv1", "conv2", "conv3"):
        P = params[lname]
        zz = {r: _tc_mm(_NT[_SRC_T[r]], P[r][0],
                        aggs=hspec[_SRC_T[r]][0],
                        dins=hspec[_SRC_T[r]][1],
                        brels=hspec[_SRC_T[r]][2],
                        dout=dout[r], out_bf16=True) for r in _RELS}
        aa1 = _sc_layer([spec("c2g", zz["c2g"])], zeros32)
        aa2 = _sc_layer([spec("go2g", zz["go2g"])], zeros32)
        ab1 = _sc_layer([spec("g2c", zz["g2c"])], zeros32)
        ab2 = _sc_layer([spec("g2go", zz["g2go"])], zeros32)
        A = {"c2g": aa1[0], "go2g": aa2[0], "g2c": ab1[0], "g2go": ab2[0]}
        hspec = {
            "gene": ([A["c2g"], A["go2g"]], [din["c2g"], din["go2g"]],
                     [P["c2g"][1], P["go2g"][1]]),
            "cell": ([A["g2c"]], [din["g2c"]], [P["g2c"][1]]),
            "gotem": ([A["g2go"]], [din["g2go"]], [P["g2go"][1]]),
        }

    return tuple(
        _tc_mm(_NT[t], params["dense"][t][0],
               aggs=hspec[t][0], dins=hspec[t][1], brels=hspec[t][2],
               b_out=params["dense"][t][1])
        for t in ("gene", "cell", "gotem"))
